# baseline (speedup 1.0000x reference)
import functools

import jax
import jax.numpy as jnp
from jax import lax
from jax.experimental import pallas as pl
from jax.experimental.pallas import tpu as pltpu
from jax.experimental.pallas import tpu_sc as plsc

NUM_WORKERS = 32
NVAR = 16
LAG = 2


def _expand_kernel(n, nh, row_words, heads_per_worker):
    mesh = plsc.VectorSubcoreMesh(core_axis_name="c", subcore_axis_name="s")
    blocks_per_head = n // 8 // NVAR

    @functools.partial(
        pl.kernel,
        out_type=jax.ShapeDtypeStruct((1, nh, n, n), jnp.float32),
        mesh=mesh,
        scratch_types=[
            pltpu.VMEM((2, 8, row_words), jnp.float32),
            pltpu.SemaphoreType.DMA,
            pltpu.SemaphoreType.DMA,
        ],
    )
    def expand(padded_hbm, out_hbm, buf, sem, stage_sem):
        wid = lax.axis_index("s") * 2 + lax.axis_index("c")
        v = wid & (NVAR - 1)
        half = wid >> 4
        h0 = half * heads_per_worker

        def drain_one():
            pltpu.make_async_copy(
                padded_hbm.at[0, 0, :, pl.ds(0, n)],
                buf.at[0, :, pl.ds(0, n)], sem).wait()

        pltpu.async_copy(padded_hbm.at[v, h0], buf.at[0], stage_sem).wait()

        for hh in range(heads_per_worker):
            h = h0 + hh
            slot = hh % 2
            if hh + 1 < heads_per_worker:
                nxt = pltpu.async_copy(padded_hbm.at[v, h + 1],
                                       buf.at[1 - slot], stage_sem)

            def b_body(B, carry, h=h, slot=slot):
                i0 = pl.multiple_of(128 * B + 8 * v, 8)
                col = pl.multiple_of(2048 - 128 * B, 128)
                pltpu.async_copy(buf.at[slot, :, pl.ds(col, n)],
                                 out_hbm.at[0, h, pl.ds(i0, 8), :], sem)

                @pl.when(B >= LAG)
                def _drain_prev():
                    drain_one()

                return carry

            lax.fori_loop(0, blocks_per_head, b_body, 0)
            for _ in range(LAG):
                drain_one()
            if hh + 1 < heads_per_worker:
                nxt.wait()

    return expand


def kernel(q_len, k_len, bias_table):
    t_rows, nh = bias_table.shape
    n = (t_rows + 1) // 2
    assert n % (8 * NVAR) == 0 and nh % (NUM_WORKERS // NVAR) == 0
    heads_per_worker = nh // (NUM_WORKERS // NVAR)
    assert n // 8 // NVAR >= LAG

    base = jnp.asarray(k_len, jnp.int32) - n + jnp.asarray(q_len, jnp.int32) - 1
    u = jnp.arange(2 * n - 1, dtype=jnp.int32)
    ext_idx = jnp.clip(u - (n - 1) + base, 0, t_rows - 1)
    ext_t = bias_table[ext_idx].T

    ext_len = 2 * n - 1
    row_words = ((ext_len + 8 * NVAR) + 127) // 128 * 128
    delta = 8 * jnp.arange(NVAR, dtype=jnp.int32)[:, None, None] + 1
    r_sub = jnp.arange(8, dtype=jnp.int32)[None, :, None]
    cols = jnp.arange(row_words, dtype=jnp.int32)[None, None, :]
    src = cols - r_sub - delta
    valid = (src >= 0) & (src < ext_len)
    gathered = ext_t[:, jnp.clip(src, 0, ext_len - 1)]
    padded = jnp.where(valid[None], gathered, 0.0).transpose(1, 0, 2, 3)

    expand = _expand_kernel(n, nh, row_words, heads_per_worker)
    return expand(padded)


# speedup vs baseline: 3.3521x; 3.3521x over previous
import functools

import jax
import jax.numpy as jnp
from jax import lax
from jax.experimental import pallas as pl
from jax.experimental.pallas import tpu as pltpu
from jax.experimental.pallas import tpu_sc as plsc

NUM_WORKERS = 32
NVAR = 16
LAG = 2


def _expand_kernel(n, nh, row_words, heads_per_worker):
    mesh = plsc.VectorSubcoreMesh(core_axis_name="c", subcore_axis_name="s")
    blocks_per_head = n // 8 // NVAR

    @functools.partial(
        pl.kernel,
        out_type=jax.ShapeDtypeStruct((1, nh, n, n), jnp.float32),
        mesh=mesh,
        scratch_types=[
            pltpu.VMEM((2, 8, row_words), jnp.float32),
            pltpu.SemaphoreType.DMA,
            pltpu.SemaphoreType.DMA,
        ],
    )
    def expand(padded_hbm, out_hbm, buf, sem, stage_sem):
        wid = lax.axis_index("s") * 2 + lax.axis_index("c")
        v = wid & (NVAR - 1)
        half = wid >> 4
        h0 = half * heads_per_worker

        def drain_one():
            pltpu.make_async_copy(
                padded_hbm.at[0, 0, :, pl.ds(0, n)],
                buf.at[0, :, pl.ds(0, n)], sem).wait()

        pltpu.async_copy(padded_hbm.at[v, h0], buf.at[0], stage_sem).wait()

        for hh in range(heads_per_worker):
            h = h0 + hh
            slot = hh % 2
            if hh + 1 < heads_per_worker:
                nxt = pltpu.async_copy(padded_hbm.at[v, h + 1],
                                       buf.at[1 - slot], stage_sem)

            def b_body(B, carry, h=h, slot=slot):
                i0 = pl.multiple_of(128 * B + 8 * v, 8)
                col = pl.multiple_of(2048 - 128 * B, 128)
                pltpu.async_copy(buf.at[slot, :, pl.ds(col, n)],
                                 out_hbm.at[0, h, pl.ds(i0, 8), :], sem)

                @pl.when(B >= LAG)
                def _drain_prev():
                    drain_one()

                return carry

            lax.fori_loop(0, blocks_per_head, b_body, 0)
            for _ in range(LAG):
                drain_one()
            if hh + 1 < heads_per_worker:
                nxt.wait()

    return expand


def kernel(q_len, k_len, bias_table):
    t_rows, nh = bias_table.shape
    n = (t_rows + 1) // 2
    assert n % (8 * NVAR) == 0 and nh % (NUM_WORKERS // NVAR) == 0
    heads_per_worker = nh // (NUM_WORKERS // NVAR)
    assert n // 8 // NVAR >= LAG

    base = jnp.asarray(k_len, jnp.int32) - n + jnp.asarray(q_len, jnp.int32) - 1
    u = jnp.arange(2 * n - 1, dtype=jnp.int32)
    ext_idx = jnp.clip(u - (n - 1) + base, 0, t_rows - 1)
    ext_t = bias_table[ext_idx].T

    ext_len = 2 * n - 1
    row_words = ((ext_len + 8 * NVAR) + 127) // 128 * 128
    delta = 8 * jnp.arange(NVAR, dtype=jnp.int32)[:, None, None] + 1
    r_sub = jnp.arange(8, dtype=jnp.int32)[None, :, None]
    cols = jnp.arange(row_words, dtype=jnp.int32)[None, None, :]
    src = cols - r_sub - delta
    valid = (src >= 0) & (src < ext_len)
    idx = jnp.clip(src, 0, ext_len - 1)
    idx_b = jnp.broadcast_to(idx[:, None], (NVAR, nh, 8, row_words))
    ext_b = jnp.broadcast_to(ext_t[None, :, None], (NVAR, nh, 8, ext_len))
    gathered = jnp.take_along_axis(ext_b, idx_b, axis=3)
    padded = jnp.where(valid[:, None], gathered, 0.0)

    expand = _expand_kernel(n, nh, row_words, heads_per_worker)
    return expand(padded)


# speedup vs baseline: 5.1913x; 1.5487x over previous
import functools

import jax
import jax.numpy as jnp
from jax import lax
from jax.experimental import pallas as pl
from jax.experimental.pallas import tpu as pltpu
from jax.experimental.pallas import tpu_sc as plsc

NUM_WORKERS = 32
NVAR = 16
LAG = 2
G_ROW = 4224


def _expand_kernel(n, nh, heads_per_worker):
    mesh = plsc.VectorSubcoreMesh(core_axis_name="c", subcore_axis_name="s")
    blocks_per_head = n // 8 // NVAR

    @functools.partial(
        pl.kernel,
        out_type=jax.ShapeDtypeStruct((1, nh, n, n), jnp.float32),
        mesh=mesh,
        scratch_types=[
            pltpu.VMEM((2, 8, G_ROW), jnp.float32),
            pltpu.SemaphoreType.DMA,
            pltpu.SemaphoreType.DMA,
        ],
    )
    def expand(g_hbm, out_hbm, buf, sem, stage_sem):
        wid = lax.axis_index("s") * 2 + lax.axis_index("c")
        v = wid & (NVAR - 1)
        half = wid >> 4
        h0 = half * heads_per_worker

        def drain_one():
            pltpu.make_async_copy(
                g_hbm.at[0, pl.ds(0, 8), pl.ds(0, n)],
                buf.at[0, :, pl.ds(0, n)], sem).wait()

        def stage(h, slot):
            return pltpu.async_copy(
                g_hbm.at[h, pl.ds(pl.multiple_of(8 * v, 8), 8), :],
                buf.at[slot], stage_sem)

        stage(h0, 0).wait()

        for hh in range(heads_per_worker):
            h = h0 + hh
            slot = hh % 2
            if hh + 1 < heads_per_worker:
                nxt = stage(h + 1, 1 - slot)

            def b_body(B, carry, h=h, slot=slot):
                i0 = pl.multiple_of(128 * B + 8 * v, 8)
                col = pl.multiple_of(2048 - 128 * B, 128)
                pltpu.async_copy(buf.at[slot, :, pl.ds(col, n)],
                                 out_hbm.at[0, h, pl.ds(i0, 8), :], sem)

                @pl.when(B >= LAG)
                def _drain_prev():
                    drain_one()

                return carry

            lax.fori_loop(0, blocks_per_head, b_body, 0)
            for _ in range(LAG):
                drain_one()
            if hh + 1 < heads_per_worker:
                nxt.wait()

    return expand


def kernel(q_len, k_len, bias_table):
    t_rows, nh = bias_table.shape
    n = (t_rows + 1) // 2
    assert n % (8 * NVAR) == 0 and nh % (NUM_WORKERS // NVAR) == 0
    heads_per_worker = nh // (NUM_WORKERS // NVAR)
    assert n // 8 // NVAR >= LAG

    base = jnp.asarray(k_len, jnp.int32) - n + jnp.asarray(q_len, jnp.int32) - 1
    u = jnp.arange(2 * n - 1, dtype=jnp.int32)
    ext_idx = jnp.clip(u - (n - 1) + base, 0, t_rows - 1)
    ext_t = bias_table[ext_idx].T

    ext_len = 2 * n - 1
    p_len = G_ROW + 1
    assert G_ROW - ext_len >= 128
    z = jnp.pad(ext_t, ((0, 0), (1, p_len - ext_len - 1)))
    n_full, rem = divmod(128 * G_ROW, p_len)
    flat = jnp.concatenate([z] * n_full + [z[:, :rem]], axis=1)
    g3 = flat.reshape(nh, 128, G_ROW)

    expand = _expand_kernel(n, nh, heads_per_worker)
    return expand(g3)


# speedup vs baseline: 9.6288x; 1.8548x over previous
import functools

import jax
import jax.numpy as jnp
from jax import lax
from jax.experimental import pallas as pl
from jax.experimental.pallas import tpu as pltpu
from jax.experimental.pallas import tpu_sc as plsc

NUM_WORKERS = 32
NVAR = 16
LAG = 2
G_ROW = 4224


def _expand_kernel(n, nh, heads_per_worker):
    mesh = plsc.VectorSubcoreMesh(core_axis_name="c", subcore_axis_name="s")
    blocks_per_head = n // 8 // NVAR

    @functools.partial(
        pl.kernel,
        out_type=jax.ShapeDtypeStruct((1, nh, n, n), jnp.float32),
        mesh=mesh,
        scratch_types=[
            pltpu.VMEM((2, 8, G_ROW), jnp.float32),
            pltpu.SemaphoreType.DMA,
            pltpu.SemaphoreType.DMA,
        ],
    )
    def expand(g_hbm, out_hbm, buf, sem, stage_sem):
        wid = lax.axis_index("s") * 2 + lax.axis_index("c")
        v = wid & (NVAR - 1)
        half = wid >> 4
        h0 = half * heads_per_worker

        def drain_one():
            pltpu.make_async_copy(
                g_hbm.at[0, pl.ds(0, 8), pl.ds(0, n)],
                buf.at[0, :, pl.ds(0, n)], sem).wait()

        def stage(h, slot):
            return pltpu.async_copy(
                g_hbm.at[h, pl.ds(pl.multiple_of(8 * v, 8), 8), :],
                buf.at[slot], stage_sem)

        stage(h0, 0).wait()

        for hh in range(heads_per_worker):
            h = h0 + hh
            slot = hh % 2
            if hh + 1 < heads_per_worker:
                nxt = stage(h + 1, 1 - slot)

            def b_body(B, carry, h=h, slot=slot):
                i0 = pl.multiple_of(128 * B + 8 * v, 8)
                col = pl.multiple_of(2048 - 128 * B, 128)
                pltpu.async_copy(buf.at[slot, :, pl.ds(col, n)],
                                 out_hbm.at[0, h, pl.ds(i0, 8), :], sem)

                @pl.when(B >= LAG)
                def _drain_prev():
                    drain_one()

                return carry

            lax.fori_loop(0, blocks_per_head, b_body, 0)
            for _ in range(LAG):
                drain_one()
            if hh + 1 < heads_per_worker:
                nxt.wait()

    return expand


def kernel(q_len, k_len, bias_table):
    t_rows, nh = bias_table.shape
    n = (t_rows + 1) // 2
    assert n % (8 * NVAR) == 0 and nh % (NUM_WORKERS // NVAR) == 0
    heads_per_worker = nh // (NUM_WORKERS // NVAR)
    assert n // 8 // NVAR >= LAG

    base = jnp.asarray(k_len, jnp.int32) - n + jnp.asarray(q_len, jnp.int32) - 1
    u = jnp.arange(2 * n - 1, dtype=jnp.int32)
    ext_idx = jnp.clip(u - (n - 1) + base, 0, t_rows - 1)
    ext_t = bias_table[ext_idx].T

    ext_len = 2 * n - 1
    p_len = G_ROW + 1
    assert G_ROW - ext_len >= 128
    z = jnp.pad(ext_t, ((0, 0), (1, p_len - ext_len - 1)))
    tiled = jnp.broadcast_to(z[:, None, :], (nh, 128, p_len))
    flat = tiled.reshape(nh, 128 * p_len)[:, :128 * G_ROW]
    g3 = flat.reshape(nh, 128, G_ROW)

    expand = _expand_kernel(n, nh, heads_per_worker)
    return expand(g3)
